# Initial kernel scaffold; baseline (speedup 1.0000x reference)
import jax
import jax.numpy as jnp
from jax import lax
from jax.experimental import pallas as pl
from jax.experimental.pallas import tpu as pltpu
from jax.experimental.pallas import tpu_sc as plsc

NC = 2
NS = 16
NW = NC * NS
LANES = 16
CHUNK = 128
GROUPS = 32
EPS = 1e-5


def _propagate(support, esrc, etgt, ef, n_nodes, d, nchunk):
    rows_per_tile = n_nodes // NS
    zrows = 125
    mesh = plsc.VectorSubcoreMesh(core_axis_name="c", subcore_axis_name="s",
                                  num_cores=NC, num_subcores=NS)

    def body(support_hbm, esrc_hbm, etgt_hbm, ef_hbm, out_hbm,
             esrc_v, etgt_v, ef_v, rows_v, zbuf, agg_sh, sem0, sem1):
        cid = lax.axis_index("c")
        sid = lax.axis_index("s")
        wid = cid * NS + sid
        sems = (sem0, sem1)

        pltpu.sync_copy(esrc_hbm.at[wid], esrc_v)
        pltpu.sync_copy(etgt_hbm.at[wid], etgt_v)
        pltpu.sync_copy(ef_hbm.at[wid], ef_v)

        zero = jnp.zeros((LANES,), jnp.float32)

        def zbody(r, c):
            for j in range(d // LANES):
                zbuf[r, pl.ds(j * LANES, LANES)] = zero
            return c

        lax.fori_loop(0, zrows, zbody, 0)
        base = sid * rows_per_tile
        for k in range(rows_per_tile // zrows):
            pltpu.sync_copy(zbuf, agg_sh.at[pl.ds(base + k * zrows, zrows)])
        plsc.subcore_barrier()

        def g_start(g, b):
            pltpu.async_copy(support_hbm.at[esrc_v.at[g]], rows_v.at[b], sems[b])

        def g_wait(g, b):
            pltpu.make_async_copy(support_hbm.at[esrc_v.at[g]], rows_v.at[b],
                                  sems[b]).wait()

        def scale(g, b):
            def ebody(e, c):
                s = ef_v[g, e]
                for j in range(d // LANES):
                    sl = pl.ds(j * LANES, LANES)
                    rows_v[b, e, sl] = rows_v[b, e, sl] * s
                return c

            lax.fori_loop(0, CHUNK, ebody, 0)

        g_start(0, 0)

        def outer(i, c):
            g0 = i * 2
            for b in range(2):
                g = g0 + b
                g_wait(g, b)
                nxt = g + 1
                if b == 0:
                    g_start(nxt, 1)
                else:
                    @pl.when(nxt < nchunk)
                    def _():
                        g_start(nxt, 0)
                scale(g, b)
                pltpu.sync_copy(rows_v.at[b], agg_sh.at[etgt_v.at[g]], add=True)
            return c

        lax.fori_loop(0, nchunk // 2, outer, 0)

        plsc.subcore_barrier()
        pltpu.sync_copy(agg_sh.at[pl.ds(base, rows_per_tile)],
                        out_hbm.at[cid, pl.ds(base, rows_per_tile)])

    f = pl.kernel(
        body,
        out_type=jax.ShapeDtypeStruct((NC, n_nodes, d), jnp.float32),
        mesh=mesh,
        scratch_types=[
            pltpu.VMEM((nchunk, CHUNK), jnp.int32),
            pltpu.VMEM((nchunk, CHUNK), jnp.int32),
            pltpu.VMEM((nchunk, CHUNK), jnp.float32),
            pltpu.VMEM((2, CHUNK, d), jnp.float32),
            pltpu.VMEM((zrows, d), jnp.float32),
            pltpu.VMEM_SHARED((n_nodes, d), jnp.float32),
            pltpu.SemaphoreType.DMA,
            pltpu.SemaphoreType.DMA,
        ],
    )
    return f(support, esrc, etgt, ef)


def _matmul(x, w, bm=1000):
    n, kdim = x.shape
    m = w.shape[1]

    def mk(x_ref, w_ref, o_ref):
        o_ref[...] = jnp.dot(x_ref[...], w_ref[...],
                             preferred_element_type=jnp.float32)

    return pl.pallas_call(
        mk,
        grid=(n // bm,),
        in_specs=[pl.BlockSpec((bm, kdim), lambda i: (i, 0)),
                  pl.BlockSpec((kdim, m), lambda i: (0, 0))],
        out_specs=pl.BlockSpec((bm, m), lambda i: (i, 0)),
        out_shape=jax.ShapeDtypeStruct((n, m), jnp.float32),
    )(x, w)


def _norm_mm(p0, p1, b0, gamma, beta, x, w2, gm, gmt, bm=1000):
    n, c = x.shape
    m = w2.shape[1]
    inv_gs = jnp.float32(GROUPS / c)

    def fk(p0_ref, p1_ref, b0_ref, g_ref, be_ref, x_ref, w2_ref, gm_ref,
           gmt_ref, o_ref):
        t = jnp.maximum(p0_ref[...] + p1_ref[...] + b0_ref[...], 0.0)
        gmat = gm_ref[...]
        gmatt = gmt_ref[...]
        m32 = jnp.dot(t, gmat, preferred_element_type=jnp.float32) * inv_gs
        mf = jnp.dot(m32, gmatt, preferred_element_type=jnp.float32)
        dlt = t - mf
        v32 = jnp.dot(dlt * dlt, gmat,
                      preferred_element_type=jnp.float32) * inv_gs
        invf = jnp.dot(lax.rsqrt(v32 + EPS), gmatt,
                       preferred_element_type=jnp.float32)
        h = dlt * invf * g_ref[...] + be_ref[...] + x_ref[...]
        o_ref[...] = jnp.dot(h, w2_ref[...], preferred_element_type=jnp.float32)

    return pl.pallas_call(
        fk,
        grid=(n // bm,),
        in_specs=[pl.BlockSpec((bm, c), lambda i: (i, 0)),
                  pl.BlockSpec((bm, c), lambda i: (i, 0)),
                  pl.BlockSpec((1, c), lambda i: (0, 0)),
                  pl.BlockSpec((1, c), lambda i: (0, 0)),
                  pl.BlockSpec((1, c), lambda i: (0, 0)),
                  pl.BlockSpec((bm, c), lambda i: (i, 0)),
                  pl.BlockSpec((c, m), lambda i: (0, 0)),
                  pl.BlockSpec((c, GROUPS), lambda i: (0, 0)),
                  pl.BlockSpec((GROUPS, c), lambda i: (0, 0))],
        out_specs=pl.BlockSpec((bm, m), lambda i: (i, 0)),
        out_shape=jax.ShapeDtypeStruct((n, m), jnp.float32),
    )(p0, p1, b0, gamma, beta, x, w2, gm, gmt)


def _combine(p0, p1, b2, ncls, bm=1000):
    n, dpad = p0.shape

    def ck(p0_ref, p1_ref, b2_ref, o_ref):
        s = p0_ref[...] + p1_ref[...]
        o_ref[...] = s[:, :ncls] + b2_ref[...]

    return pl.pallas_call(
        ck,
        grid=(n // bm,),
        in_specs=[pl.BlockSpec((bm, dpad), lambda i: (i, 0)),
                  pl.BlockSpec((bm, dpad), lambda i: (i, 0)),
                  pl.BlockSpec((1, ncls), lambda i: (0, 0))],
        out_specs=pl.BlockSpec((bm, ncls), lambda i: (i, 0)),
        out_shape=jax.ShapeDtypeStruct((n, ncls), jnp.float32),
    )(p0, p1, b2)


def kernel(x, Esrc, Etgt, ef, W0, b0, gamma0, beta0, W2, b2):
    n, c = x.shape
    e = Esrc.shape[0]
    ncls = W2.shape[1]
    d2 = 64

    epr = CHUNK * NW
    nchunk = -(-e // epr)
    if nchunk % 2:
        nchunk += 1
    pad = nchunk * epr - e

    esrc_p = jnp.pad(Esrc, (0, pad)).reshape(NW, nchunk, CHUNK)
    etgt_p = jnp.pad(Etgt, (0, pad)).reshape(NW, nchunk, CHUNK)
    ef_p = jnp.pad(ef[:, 0], (0, pad)).reshape(NW, nchunk, CHUNK)
    w2p = jnp.pad(W2, ((0, 0), (0, d2 - ncls)))
    gm = jnp.repeat(jnp.eye(GROUPS, dtype=jnp.float32), c // GROUPS, axis=0)

    support = _matmul(x, W0)
    parts = _propagate(support, esrc_p, etgt_p, ef_p, n, c, nchunk)
    support2 = _norm_mm(parts[0], parts[1], b0.reshape(1, c),
                        gamma0.reshape(1, c), beta0.reshape(1, c),
                        x, w2p, gm, gm.T)
    parts2 = _propagate(support2, esrc_p, etgt_p, ef_p, n, d2, nchunk)
    return _combine(parts2[0], parts2[1], b2.reshape(1, ncls), ncls)


# speedup vs baseline: 3.6112x; 3.6112x over previous
import jax
import jax.numpy as jnp
from jax import lax
from jax.experimental import pallas as pl
from jax.experimental.pallas import tpu as pltpu
from jax.experimental.pallas import tpu_sc as plsc

NC = 2
NS = 16
NW = NC * NS
LANES = 16
CHUNK = 128
GROUPS = 32
EPS = 1e-5


def _propagate(support, esrc, etgt, ef, n_nodes, d, nblk, blk):
    rpt = (n_nodes // NS) & ~7
    tail = n_nodes - NS * rpt
    mesh = plsc.VectorSubcoreMesh(core_axis_name="c", subcore_axis_name="s",
                                  num_cores=NC, num_subcores=NS)

    def body(support_hbm, esrc_hbm, etgt_hbm, ef_hbm, out_hbm,
             esrc_v, etgt_v, ef_v, rows_v, agg_sh, sem0, sem1, semi0, semi1):
        cid = lax.axis_index("c")
        sid = lax.axis_index("s")
        wid = cid * NS + sid
        sems = (sem0, sem1)
        isems = (semi0, semi1)

        def idx_start(B, ib):
            pltpu.async_copy(esrc_hbm.at[wid, B], esrc_v.at[ib], isems[ib])
            pltpu.async_copy(etgt_hbm.at[wid, B], etgt_v.at[ib], isems[ib])
            pltpu.async_copy(ef_hbm.at[wid, B], ef_v.at[ib], isems[ib])

        def idx_wait(B, ib):
            pltpu.make_async_copy(esrc_hbm.at[wid, B], esrc_v.at[ib],
                                  isems[ib]).wait()
            pltpu.make_async_copy(etgt_hbm.at[wid, B], etgt_v.at[ib],
                                  isems[ib]).wait()
            pltpu.make_async_copy(ef_hbm.at[wid, B], ef_v.at[ib],
                                  isems[ib]).wait()

        idx_start(0, 0)

        zero = jnp.zeros((LANES,), jnp.float32)

        def zbody(r, c):
            for j in range(d // LANES):
                rows_v[0, r, pl.ds(j * LANES, LANES)] = zero
            return c

        lax.fori_loop(0, CHUNK, zbody, 0)
        base = sid * rpt
        off, rem = 0, rpt
        while rem > 0:
            ln = min(CHUNK, rem)
            pltpu.sync_copy(rows_v.at[0].at[pl.ds(0, ln)],
                            agg_sh.at[pl.ds(base + off, ln)])
            off, rem = off + ln, rem - ln
        if tail:
            @pl.when(sid == NS - 1)
            def _():
                pltpu.sync_copy(rows_v.at[0].at[pl.ds(0, tail)],
                                agg_sh.at[pl.ds(NS * rpt, tail)])
        plsc.subcore_barrier()

        def g_start(ib, j, b):
            pltpu.async_copy(support_hbm.at[esrc_v.at[ib, j]], rows_v.at[b],
                             sems[b])

        def g_wait(ib, j, b):
            pltpu.make_async_copy(support_hbm.at[esrc_v.at[ib, j]],
                                  rows_v.at[b], sems[b]).wait()

        def scale(ib, j, b):
            def ebody(e0, c):
                ef16 = ef_v[ib, j, pl.ds(e0 * LANES, LANES)]
                for k in range(LANES):
                    s = ef16[k]
                    e = e0 * LANES + k
                    for jj in range(d // LANES):
                        sl = pl.ds(jj * LANES, LANES)
                        rows_v[b, e, sl] = rows_v[b, e, sl] * s
                return c

            lax.fori_loop(0, CHUNK // LANES, ebody, 0)

        for B in range(nblk):
            ib = B % 2
            idx_wait(B, ib)
            if B + 1 < nblk:
                idx_start(B + 1, 1 - ib)
            g_start(ib, 0, 0)

            def inner(i, c, ib=ib):
                for b in range(2):
                    j = i * 2 + b
                    g_wait(ib, j, b)
                    if b == 0:
                        g_start(ib, j + 1, 1)
                    else:
                        @pl.when(j + 1 < blk)
                        def _():
                            g_start(ib, j + 1, 0)
                    scale(ib, j, b)
                    pltpu.sync_copy(rows_v.at[b],
                                    agg_sh.at[etgt_v.at[ib, j]], add=True)
                return c

            lax.fori_loop(0, blk // 2, inner, 0)

        plsc.subcore_barrier()
        pltpu.sync_copy(agg_sh.at[pl.ds(base, rpt)],
                        out_hbm.at[cid, pl.ds(base, rpt)])
        if tail:
            @pl.when(sid == NS - 1)
            def _():
                pltpu.sync_copy(agg_sh.at[pl.ds(NS * rpt, tail)],
                                out_hbm.at[cid, pl.ds(NS * rpt, tail)])

    f = pl.kernel(
        body,
        out_type=jax.ShapeDtypeStruct((NC, n_nodes, d), jnp.float32),
        mesh=mesh,
        scratch_types=[
            pltpu.VMEM((2, blk, CHUNK), jnp.int32),
            pltpu.VMEM((2, blk, CHUNK), jnp.int32),
            pltpu.VMEM((2, blk, CHUNK), jnp.float32),
            pltpu.VMEM((2, CHUNK, d), jnp.float32),
            pltpu.VMEM_SHARED((n_nodes, d), jnp.float32),
            pltpu.SemaphoreType.DMA,
            pltpu.SemaphoreType.DMA,
            pltpu.SemaphoreType.DMA,
            pltpu.SemaphoreType.DMA,
        ],
        compiler_params=pltpu.CompilerParams(use_tc_tiling_on_sc=False),
    )
    return f(support, esrc, etgt, ef)


def _matmul(x, w, bm=1000):
    n, kdim = x.shape
    m = w.shape[1]

    def mk(x_ref, w_ref, o_ref):
        o_ref[...] = jnp.dot(x_ref[...], w_ref[...],
                             preferred_element_type=jnp.float32)

    return pl.pallas_call(
        mk,
        grid=(n // bm,),
        in_specs=[pl.BlockSpec((bm, kdim), lambda i: (i, 0)),
                  pl.BlockSpec((kdim, m), lambda i: (0, 0))],
        out_specs=pl.BlockSpec((bm, m), lambda i: (i, 0)),
        out_shape=jax.ShapeDtypeStruct((n, m), jnp.float32),
    )(x, w)


def _norm_mm(p0, p1, b0, gamma, beta, x, w2, gm, gmt, bm=1000):
    n, c = x.shape
    m = w2.shape[1]
    inv_gs = float(GROUPS) / float(c)

    def fk(p0_ref, p1_ref, b0_ref, g_ref, be_ref, x_ref, w2_ref, gm_ref,
           gmt_ref, o_ref):
        t = jnp.maximum(p0_ref[...] + p1_ref[...] + b0_ref[...], 0.0)
        gmat = gm_ref[...]
        gmatt = gmt_ref[...]
        m32 = jnp.dot(t, gmat, preferred_element_type=jnp.float32) * inv_gs
        mf = jnp.dot(m32, gmatt, preferred_element_type=jnp.float32)
        dlt = t - mf
        v32 = jnp.dot(dlt * dlt, gmat,
                      preferred_element_type=jnp.float32) * inv_gs
        invf = jnp.dot(lax.rsqrt(v32 + EPS), gmatt,
                       preferred_element_type=jnp.float32)
        h = dlt * invf * g_ref[...] + be_ref[...] + x_ref[...]
        o_ref[...] = jnp.dot(h, w2_ref[...], preferred_element_type=jnp.float32)

    return pl.pallas_call(
        fk,
        grid=(n // bm,),
        in_specs=[pl.BlockSpec((bm, c), lambda i: (i, 0)),
                  pl.BlockSpec((bm, c), lambda i: (i, 0)),
                  pl.BlockSpec((1, c), lambda i: (0, 0)),
                  pl.BlockSpec((1, c), lambda i: (0, 0)),
                  pl.BlockSpec((1, c), lambda i: (0, 0)),
                  pl.BlockSpec((bm, c), lambda i: (i, 0)),
                  pl.BlockSpec((c, m), lambda i: (0, 0)),
                  pl.BlockSpec((c, GROUPS), lambda i: (0, 0)),
                  pl.BlockSpec((GROUPS, c), lambda i: (0, 0))],
        out_specs=pl.BlockSpec((bm, m), lambda i: (i, 0)),
        out_shape=jax.ShapeDtypeStruct((n, m), jnp.float32),
    )(p0, p1, b0, gamma, beta, x, w2, gm, gmt)


def _combine(p0, p1, b2, ncls, bm=1000):
    n, dpad = p0.shape

    def ck(p0_ref, p1_ref, b2_ref, o_ref):
        s = p0_ref[...] + p1_ref[...]
        o_ref[...] = s[:, :ncls] + b2_ref[...]

    return pl.pallas_call(
        ck,
        grid=(n // bm,),
        in_specs=[pl.BlockSpec((bm, dpad), lambda i: (i, 0)),
                  pl.BlockSpec((bm, dpad), lambda i: (i, 0)),
                  pl.BlockSpec((1, ncls), lambda i: (0, 0))],
        out_specs=pl.BlockSpec((bm, ncls), lambda i: (i, 0)),
        out_shape=jax.ShapeDtypeStruct((n, ncls), jnp.float32),
    )(p0, p1, b2)


def kernel(x, Esrc, Etgt, ef, W0, b0, gamma0, beta0, W2, b2):
    n, c = x.shape
    e = Esrc.shape[0]
    ncls = W2.shape[1]
    d2 = 64

    blk = 16
    epr = CHUNK * NW
    nchunk = -(-e // (epr * blk)) * blk
    nblk = nchunk // blk
    pad = nchunk * epr - e

    esrc_p = jnp.pad(Esrc, (0, pad)).reshape(NW, nblk, blk, CHUNK)
    etgt_p = jnp.pad(Etgt, (0, pad)).reshape(NW, nblk, blk, CHUNK)
    ef_p = jnp.pad(ef[:, 0], (0, pad)).reshape(NW, nblk, blk, CHUNK)
    w2p = jnp.pad(W2, ((0, 0), (0, d2 - ncls)))
    gm = jnp.repeat(jnp.eye(GROUPS, dtype=jnp.float32), c // GROUPS, axis=0)

    support = _matmul(x, W0)
    parts = _propagate(support, esrc_p, etgt_p, ef_p, n, c, nblk, blk)
    support2 = _norm_mm(parts[0], parts[1], b0.reshape(1, c),
                        gamma0.reshape(1, c), beta0.reshape(1, c),
                        x, w2p, gm, gm.T)
    parts2 = _propagate(support2, esrc_p, etgt_p, ef_p, n, d2, nblk, blk)
    return _combine(parts2[0], parts2[1], b2.reshape(1, ncls), ncls)


# speedup vs baseline: 4.1220x; 1.1414x over previous
import jax
import jax.numpy as jnp
from jax import lax
from jax.experimental import pallas as pl
from jax.experimental.pallas import tpu as pltpu
from jax.experimental.pallas import tpu_sc as plsc

NC = 2
NS = 16
NW = NC * NS
LANES = 16
CHUNK = 128
GROUPS = 32
EPS = 1e-5


def _propagate(support, esrc, etgt, ef, n_nodes, d, nblk, blk):
    rpt = (n_nodes // NS) & ~7
    tail = n_nodes - NS * rpt
    mesh = plsc.VectorSubcoreMesh(core_axis_name="c", subcore_axis_name="s",
                                  num_cores=NC, num_subcores=NS)

    def body(support_hbm, esrc_hbm, etgt_hbm, ef_hbm, out_hbm,
             esrc_v, etgt_v, ef_v, rows_v, agg_sh, sem0, sem1, semi0, semi1):
        cid = lax.axis_index("c")
        sid = lax.axis_index("s")
        wid = cid * NS + sid
        sems = (sem0, sem1)
        isems = (semi0, semi1)

        def idx_start(B, ib):
            pltpu.async_copy(esrc_hbm.at[wid, B], esrc_v.at[ib], isems[ib])
            pltpu.async_copy(etgt_hbm.at[wid, B], etgt_v.at[ib], isems[ib])
            pltpu.async_copy(ef_hbm.at[wid, B], ef_v.at[ib], isems[ib])

        def idx_wait(B, ib):
            pltpu.make_async_copy(esrc_hbm.at[wid, B], esrc_v.at[ib],
                                  isems[ib]).wait()
            pltpu.make_async_copy(etgt_hbm.at[wid, B], etgt_v.at[ib],
                                  isems[ib]).wait()
            pltpu.make_async_copy(ef_hbm.at[wid, B], ef_v.at[ib],
                                  isems[ib]).wait()

        idx_start(0, 0)

        zero = jnp.zeros((LANES,), jnp.float32)

        def zbody(r, c):
            for j in range(d // LANES):
                rows_v[0, r, pl.ds(j * LANES, LANES)] = zero
            return c

        lax.fori_loop(0, CHUNK, zbody, 0)
        base = sid * rpt
        off, rem = 0, rpt
        while rem > 0:
            ln = min(CHUNK, rem)
            pltpu.sync_copy(rows_v.at[0].at[pl.ds(0, ln)],
                            agg_sh.at[pl.ds(base + off, ln)])
            off, rem = off + ln, rem - ln
        if tail:
            @pl.when(sid == NS - 1)
            def _():
                pltpu.sync_copy(rows_v.at[0].at[pl.ds(0, tail)],
                                agg_sh.at[pl.ds(NS * rpt, tail)])
        plsc.subcore_barrier()

        def g_start(ib, j, b):
            pltpu.async_copy(support_hbm.at[esrc_v.at[ib, j]], rows_v.at[b],
                             sems[b])

        def g_wait(ib, j, b):
            pltpu.make_async_copy(support_hbm.at[esrc_v.at[ib, j]],
                                  rows_v.at[b], sems[b]).wait()

        def scale(ib, j, b):
            def ebody(e0, c):
                ef16 = ef_v[ib, j, pl.ds(e0 * LANES, LANES)]
                for k in range(LANES):
                    s = ef16[k]
                    e = e0 * LANES + k
                    for jj in range(d // LANES):
                        sl = pl.ds(jj * LANES, LANES)
                        rows_v[b, e, sl] = rows_v[b, e, sl] * s
                return c

            lax.fori_loop(0, CHUNK // LANES, ebody, 0)

        for B in range(nblk):
            ib = B % 2
            idx_wait(B, ib)
            if B + 1 < nblk:
                idx_start(B + 1, 1 - ib)
            g_start(ib, 0, 0)

            def inner(i, c, ib=ib):
                for b in range(2):
                    j = i * 2 + b
                    g_wait(ib, j, b)
                    if b == 0:
                        g_start(ib, j + 1, 1)
                    else:
                        @pl.when(j + 1 < blk)
                        def _():
                            g_start(ib, j + 1, 0)
                    scale(ib, j, b)
                    pltpu.sync_copy(rows_v.at[b],
                                    agg_sh.at[etgt_v.at[ib, j]], add=True)
                return c

            lax.fori_loop(0, blk // 2, inner, 0)

        plsc.subcore_barrier()
        pltpu.sync_copy(agg_sh.at[pl.ds(base, rpt)],
                        out_hbm.at[cid, pl.ds(base, rpt)])
        if tail:
            @pl.when(sid == NS - 1)
            def _():
                pltpu.sync_copy(agg_sh.at[pl.ds(NS * rpt, tail)],
                                out_hbm.at[cid, pl.ds(NS * rpt, tail)])

    f = pl.kernel(
        body,
        out_type=jax.ShapeDtypeStruct((NC, n_nodes, d), jnp.float32),
        mesh=mesh,
        scratch_types=[
            pltpu.VMEM((2, blk, CHUNK), jnp.int32),
            pltpu.VMEM((2, blk, CHUNK), jnp.int32),
            pltpu.VMEM((2, blk, CHUNK), jnp.float32),
            pltpu.VMEM((2, CHUNK, d), jnp.float32),
            pltpu.VMEM_SHARED((n_nodes, d), jnp.float32),
            pltpu.SemaphoreType.DMA,
            pltpu.SemaphoreType.DMA,
            pltpu.SemaphoreType.DMA,
            pltpu.SemaphoreType.DMA,
        ],
        compiler_params=pltpu.CompilerParams(use_tc_tiling_on_sc=False),
    )
    return f(support, esrc, etgt, ef)


def _propagate_flat(support, esrc, etgt, ef, n_nodes, d, nchunk, nbuf=4):
    rpt = (n_nodes // NS) & ~7
    tail = n_nodes - NS * rpt
    mesh = plsc.VectorSubcoreMesh(core_axis_name="c", subcore_axis_name="s",
                                  num_cores=NC, num_subcores=NS)

    def body(support_hbm, esrc_hbm, etgt_hbm, ef_hbm, out_hbm,
             esrc_v, etgt_v, ef_v, rows_v, agg_sh, *sems):
        cid = lax.axis_index("c")
        sid = lax.axis_index("s")
        wid = cid * NS + sid

        pltpu.sync_copy(esrc_hbm.at[wid], esrc_v)
        pltpu.sync_copy(etgt_hbm.at[wid], etgt_v)
        pltpu.sync_copy(ef_hbm.at[wid], ef_v)

        zero = jnp.zeros((LANES,), jnp.float32)

        def zbody(r, c):
            for j in range(d // LANES):
                rows_v[0, r, pl.ds(j * LANES, LANES)] = zero
            return c

        lax.fori_loop(0, CHUNK, zbody, 0)
        base = sid * rpt
        off, rem = 0, rpt
        while rem > 0:
            ln = min(CHUNK, rem)
            pltpu.sync_copy(rows_v.at[0].at[pl.ds(0, ln)],
                            agg_sh.at[pl.ds(base + off, ln)])
            off, rem = off + ln, rem - ln
        if tail:
            @pl.when(sid == NS - 1)
            def _():
                pltpu.sync_copy(rows_v.at[0].at[pl.ds(0, tail)],
                                agg_sh.at[pl.ds(NS * rpt, tail)])
        plsc.subcore_barrier()

        def g_start(j, b):
            pltpu.async_copy(support_hbm.at[esrc_v.at[j]], rows_v.at[b],
                             sems[b])

        def g_wait(j, b):
            pltpu.make_async_copy(support_hbm.at[esrc_v.at[j]],
                                  rows_v.at[b], sems[b]).wait()

        def scale(j, b):
            def ebody(e0, c):
                ef16 = ef_v[j, pl.ds(e0 * LANES, LANES)]
                for k in range(LANES):
                    s = ef16[k]
                    e = e0 * LANES + k
                    for jj in range(d // LANES):
                        sl = pl.ds(jj * LANES, LANES)
                        rows_v[b, e, sl] = rows_v[b, e, sl] * s
                return c

            lax.fori_loop(0, CHUNK // LANES, ebody, 0)

        for j in range(nbuf - 1):
            g_start(j, j)

        def inner(i, c):
            for b in range(nbuf):
                j = i * nbuf + b
                g_wait(j, b)
                nxt = j + nbuf - 1

                @pl.when(nxt < nchunk)
                def _():
                    g_start(nxt, (b + nbuf - 1) % nbuf)

                scale(j, b)
                pltpu.sync_copy(rows_v.at[b], agg_sh.at[etgt_v.at[j]],
                                add=True)
            return c

        lax.fori_loop(0, nchunk // nbuf, inner, 0)

        plsc.subcore_barrier()
        pltpu.sync_copy(agg_sh.at[pl.ds(base, rpt)],
                        out_hbm.at[cid, pl.ds(base, rpt)])
        if tail:
            @pl.when(sid == NS - 1)
            def _():
                pltpu.sync_copy(agg_sh.at[pl.ds(NS * rpt, tail)],
                                out_hbm.at[cid, pl.ds(NS * rpt, tail)])

    f = pl.kernel(
        body,
        out_type=jax.ShapeDtypeStruct((NC, n_nodes, d), jnp.float32),
        mesh=mesh,
        scratch_types=[
            pltpu.VMEM((nchunk, CHUNK), jnp.int32),
            pltpu.VMEM((nchunk, CHUNK), jnp.int32),
            pltpu.VMEM((nchunk, CHUNK), jnp.float32),
            pltpu.VMEM((nbuf, CHUNK, d), jnp.float32),
            pltpu.VMEM_SHARED((n_nodes, d), jnp.float32),
        ] + [pltpu.SemaphoreType.DMA] * nbuf,
        compiler_params=pltpu.CompilerParams(use_tc_tiling_on_sc=False),
    )
    return f(support, esrc, etgt, ef)


def _matmul(x, w, bm=1000):
    n, kdim = x.shape
    m = w.shape[1]

    def mk(x_ref, w_ref, o_ref):
        o_ref[...] = jnp.dot(x_ref[...], w_ref[...],
                             preferred_element_type=jnp.float32)

    return pl.pallas_call(
        mk,
        grid=(n // bm,),
        in_specs=[pl.BlockSpec((bm, kdim), lambda i: (i, 0)),
                  pl.BlockSpec((kdim, m), lambda i: (0, 0))],
        out_specs=pl.BlockSpec((bm, m), lambda i: (i, 0)),
        out_shape=jax.ShapeDtypeStruct((n, m), jnp.float32),
    )(x, w)


def _norm_mm(p0, p1, b0, gamma, beta, x, w2, gm, gmt, bm=1000):
    n, c = x.shape
    m = w2.shape[1]
    inv_gs = float(GROUPS) / float(c)

    def fk(p0_ref, p1_ref, b0_ref, g_ref, be_ref, x_ref, w2_ref, gm_ref,
           gmt_ref, o_ref):
        t = jnp.maximum(p0_ref[...] + p1_ref[...] + b0_ref[...], 0.0)
        gmat = gm_ref[...]
        gmatt = gmt_ref[...]
        m32 = jnp.dot(t, gmat, preferred_element_type=jnp.float32) * inv_gs
        mf = jnp.dot(m32, gmatt, preferred_element_type=jnp.float32)
        dlt = t - mf
        v32 = jnp.dot(dlt * dlt, gmat,
                      preferred_element_type=jnp.float32) * inv_gs
        invf = jnp.dot(lax.rsqrt(v32 + EPS), gmatt,
                       preferred_element_type=jnp.float32)
        h = dlt * invf * g_ref[...] + be_ref[...] + x_ref[...]
        o_ref[...] = jnp.dot(h, w2_ref[...], preferred_element_type=jnp.float32)

    return pl.pallas_call(
        fk,
        grid=(n // bm,),
        in_specs=[pl.BlockSpec((bm, c), lambda i: (i, 0)),
                  pl.BlockSpec((bm, c), lambda i: (i, 0)),
                  pl.BlockSpec((1, c), lambda i: (0, 0)),
                  pl.BlockSpec((1, c), lambda i: (0, 0)),
                  pl.BlockSpec((1, c), lambda i: (0, 0)),
                  pl.BlockSpec((bm, c), lambda i: (i, 0)),
                  pl.BlockSpec((c, m), lambda i: (0, 0)),
                  pl.BlockSpec((c, GROUPS), lambda i: (0, 0)),
                  pl.BlockSpec((GROUPS, c), lambda i: (0, 0))],
        out_specs=pl.BlockSpec((bm, m), lambda i: (i, 0)),
        out_shape=jax.ShapeDtypeStruct((n, m), jnp.float32),
    )(p0, p1, b0, gamma, beta, x, w2, gm, gmt)


def _combine(p0, p1, b2, ncls, bm=1000):
    n, dpad = p0.shape

    def ck(p0_ref, p1_ref, b2_ref, o_ref):
        s = p0_ref[...] + p1_ref[...]
        o_ref[...] = s[:, :ncls] + b2_ref[...]

    return pl.pallas_call(
        ck,
        grid=(n // bm,),
        in_specs=[pl.BlockSpec((bm, dpad), lambda i: (i, 0)),
                  pl.BlockSpec((bm, dpad), lambda i: (i, 0)),
                  pl.BlockSpec((1, ncls), lambda i: (0, 0))],
        out_specs=pl.BlockSpec((bm, ncls), lambda i: (i, 0)),
        out_shape=jax.ShapeDtypeStruct((n, ncls), jnp.float32),
    )(p0, p1, b2)


def kernel(x, Esrc, Etgt, ef, W0, b0, gamma0, beta0, W2, b2):
    n, c = x.shape
    e = Esrc.shape[0]
    ncls = W2.shape[1]
    d2 = 48

    blk = 16
    epr = CHUNK * NW
    nchunk = -(-e // (epr * blk)) * blk
    nblk = nchunk // blk
    pad = nchunk * epr - e

    esrc_p = jnp.pad(Esrc, (0, pad)).reshape(NW, nblk, blk, CHUNK)
    etgt_p = jnp.pad(Etgt, (0, pad)).reshape(NW, nblk, blk, CHUNK)
    ef_p = jnp.pad(ef[:, 0], (0, pad)).reshape(NW, nblk, blk, CHUNK)
    w2p = jnp.pad(W2, ((0, 0), (0, d2 - ncls)))
    gm = jnp.repeat(jnp.eye(GROUPS, dtype=jnp.float32), c // GROUPS, axis=0)

    support = _matmul(x, W0)
    parts = _propagate(support, esrc_p, etgt_p, ef_p, n, c, nblk, blk)
    support2 = _norm_mm(parts[0], parts[1], b0.reshape(1, c),
                        gamma0.reshape(1, c), beta0.reshape(1, c),
                        x, w2p, gm, gm.T)
    parts2 = _propagate_flat(support2, esrc_p.reshape(NW, nchunk, CHUNK),
                             etgt_p.reshape(NW, nchunk, CHUNK),
                             ef_p.reshape(NW, nchunk, CHUNK), n, d2, nchunk)
    return _combine(parts2[0], parts2[1], b2.reshape(1, ncls), ncls)


# speedup vs baseline: 6.3274x; 1.5350x over previous
import jax
import jax.numpy as jnp
from jax import lax
from jax.experimental import pallas as pl
from jax.experimental.pallas import tpu as pltpu
from jax.experimental.pallas import tpu_sc as plsc

NC = 2
NS = 16
NW = NC * NS
LANES = 16
CHUNK = 128
GROUPS = 32
EPS = 1e-5


def _propagate(support, esrc, etgt, ef, n_nodes, d, nblk, blk, ch):
    rpt = (n_nodes // NS) & ~7
    tail = n_nodes - NS * rpt
    mesh = plsc.VectorSubcoreMesh(core_axis_name="c", subcore_axis_name="s",
                                  num_cores=NC, num_subcores=NS)

    def body(support_hbm, esrc_hbm, etgt_hbm, ef_hbm, out_hbm,
             esrc_v, etgt_v, ef_v, rows_v, agg_sh,
             sem0, sem1, sem2, semi0, semi1):
        cid = lax.axis_index("c")
        sid = lax.axis_index("s")
        wid = cid * NS + sid
        sems = (sem0, sem1, sem2)
        isems = (semi0, semi1)

        def idx_start(B, ib):
            pltpu.async_copy(esrc_hbm.at[wid, B], esrc_v.at[ib], isems[ib])
            pltpu.async_copy(etgt_hbm.at[wid, B], etgt_v.at[ib], isems[ib])
            pltpu.async_copy(ef_hbm.at[wid, B], ef_v.at[ib], isems[ib])

        def idx_wait(B, ib):
            pltpu.make_async_copy(esrc_hbm.at[wid, B], esrc_v.at[ib],
                                  isems[ib]).wait()
            pltpu.make_async_copy(etgt_hbm.at[wid, B], etgt_v.at[ib],
                                  isems[ib]).wait()
            pltpu.make_async_copy(ef_hbm.at[wid, B], ef_v.at[ib],
                                  isems[ib]).wait()

        idx_start(0, 0)

        zero = jnp.zeros((LANES,), jnp.float32)

        def zbody(r, c):
            for j in range(d // LANES):
                rows_v[0, r, pl.ds(j * LANES, LANES)] = zero
            return c

        lax.fori_loop(0, ch, zbody, 0)
        base = sid * rpt
        off, rem = 0, rpt
        while rem > 0:
            ln = min(ch, rem)
            pltpu.sync_copy(rows_v.at[0].at[pl.ds(0, ln)],
                            agg_sh.at[pl.ds(base + off, ln)])
            off, rem = off + ln, rem - ln
        if tail:
            @pl.when(sid == NS - 1)
            def _():
                pltpu.sync_copy(rows_v.at[0].at[pl.ds(0, tail)],
                                agg_sh.at[pl.ds(NS * rpt, tail)])
        plsc.subcore_barrier()

        def g_start(ib, j, b):
            pltpu.async_copy(support_hbm.at[esrc_v.at[ib, j]], rows_v.at[b],
                             sems[b])

        def g_wait(ib, j, b):
            pltpu.make_async_copy(support_hbm.at[esrc_v.at[ib, j]],
                                  rows_v.at[b], sems[b]).wait()

        def scale(ib, j, b):
            def ebody(e0, c):
                ef16 = ef_v[ib, j, pl.ds(e0 * LANES, LANES)]
                for k in range(LANES):
                    s = ef16[k]
                    e = e0 * LANES + k
                    for jj in range(d // LANES):
                        sl = pl.ds(jj * LANES, LANES)
                        rows_v[b, e, sl] = rows_v[b, e, sl] * s
                return c

            lax.fori_loop(0, ch // LANES, ebody, 0)

        idx_wait(0, 0)
        if nblk > 1:
            idx_start(1, 1)
        g_start(0, 0, 0)
        g_start(0, 1, 1)

        for B in range(nblk):
            ib = B % 2

            def inner(i, c, ib=ib):
                for b in range(3):
                    j = i * 3 + b
                    g_wait(ib, j, b)

                    @pl.when(j + 2 < blk)
                    def _():
                        g_start(ib, j + 2, (b + 2) % 3)

                    scale(ib, j, b)
                    pltpu.sync_copy(rows_v.at[b],
                                    agg_sh.at[etgt_v.at[ib, j]], add=True)
                return c

            lax.fori_loop(0, blk // 3, inner, 0)

            if B + 1 < nblk:
                idx_wait(B + 1, 1 - ib)
                if B + 2 < nblk:
                    idx_start(B + 2, ib)
                g_start(1 - ib, 0, 0)
                g_start(1 - ib, 1, 1)

        plsc.subcore_barrier()
        pltpu.sync_copy(agg_sh.at[pl.ds(base, rpt)],
                        out_hbm.at[cid, pl.ds(base, rpt)])
        if tail:
            @pl.when(sid == NS - 1)
            def _():
                pltpu.sync_copy(agg_sh.at[pl.ds(NS * rpt, tail)],
                                out_hbm.at[cid, pl.ds(NS * rpt, tail)])

    f = pl.kernel(
        body,
        out_type=jax.ShapeDtypeStruct((NC, n_nodes, d), jnp.float32),
        mesh=mesh,
        scratch_types=[
            pltpu.VMEM((2, blk, ch), jnp.int32),
            pltpu.VMEM((2, blk, ch), jnp.int32),
            pltpu.VMEM((2, blk, ch), jnp.float32),
            pltpu.VMEM((3, ch, d), jnp.float32),
            pltpu.VMEM_SHARED((n_nodes, d), jnp.float32),
            pltpu.SemaphoreType.DMA,
            pltpu.SemaphoreType.DMA,
            pltpu.SemaphoreType.DMA,
            pltpu.SemaphoreType.DMA,
            pltpu.SemaphoreType.DMA,
        ],
        compiler_params=pltpu.CompilerParams(use_tc_tiling_on_sc=False),
    )
    return f(support, esrc, etgt, ef)


def _propagate_flat(support, esrc, etgt, ef, n_nodes, d, nchunk, nbuf=4):
    rpt = (n_nodes // NS) & ~7
    tail = n_nodes - NS * rpt
    mesh = plsc.VectorSubcoreMesh(core_axis_name="c", subcore_axis_name="s",
                                  num_cores=NC, num_subcores=NS)

    def body(support_hbm, esrc_hbm, etgt_hbm, ef_hbm, out_hbm,
             esrc_v, etgt_v, ef_v, rows_v, agg_sh, *sems):
        cid = lax.axis_index("c")
        sid = lax.axis_index("s")
        wid = cid * NS + sid

        pltpu.sync_copy(esrc_hbm.at[wid], esrc_v)
        pltpu.sync_copy(etgt_hbm.at[wid], etgt_v)
        pltpu.sync_copy(ef_hbm.at[wid], ef_v)

        zero = jnp.zeros((LANES,), jnp.float32)

        def zbody(r, c):
            for j in range(d // LANES):
                rows_v[0, r, pl.ds(j * LANES, LANES)] = zero
            return c

        lax.fori_loop(0, CHUNK, zbody, 0)
        base = sid * rpt
        off, rem = 0, rpt
        while rem > 0:
            ln = min(CHUNK, rem)
            pltpu.sync_copy(rows_v.at[0].at[pl.ds(0, ln)],
                            agg_sh.at[pl.ds(base + off, ln)])
            off, rem = off + ln, rem - ln
        if tail:
            @pl.when(sid == NS - 1)
            def _():
                pltpu.sync_copy(rows_v.at[0].at[pl.ds(0, tail)],
                                agg_sh.at[pl.ds(NS * rpt, tail)])
        plsc.subcore_barrier()

        def g_start(j, b):
            pltpu.async_copy(support_hbm.at[esrc_v.at[j]], rows_v.at[b],
                             sems[b])

        def g_wait(j, b):
            pltpu.make_async_copy(support_hbm.at[esrc_v.at[j]],
                                  rows_v.at[b], sems[b]).wait()

        def scale(j, b):
            def ebody(e0, c):
                ef16 = ef_v[j, pl.ds(e0 * LANES, LANES)]
                for k in range(LANES):
                    s = ef16[k]
                    e = e0 * LANES + k
                    for jj in range(d // LANES):
                        sl = pl.ds(jj * LANES, LANES)
                        rows_v[b, e, sl] = rows_v[b, e, sl] * s
                return c

            lax.fori_loop(0, CHUNK // LANES, ebody, 0)

        for j in range(nbuf - 1):
            g_start(j, j)

        def inner(i, c):
            for b in range(nbuf):
                j = i * nbuf + b
                g_wait(j, b)
                nxt = j + nbuf - 1

                @pl.when(nxt < nchunk)
                def _():
                    g_start(nxt, (b + nbuf - 1) % nbuf)

                scale(j, b)
                pltpu.sync_copy(rows_v.at[b], agg_sh.at[etgt_v.at[j]],
                                add=True)
            return c

        lax.fori_loop(0, nchunk // nbuf, inner, 0)

        plsc.subcore_barrier()
        pltpu.sync_copy(agg_sh.at[pl.ds(base, rpt)],
                        out_hbm.at[cid, pl.ds(base, rpt)])
        if tail:
            @pl.when(sid == NS - 1)
            def _():
                pltpu.sync_copy(agg_sh.at[pl.ds(NS * rpt, tail)],
                                out_hbm.at[cid, pl.ds(NS * rpt, tail)])

    f = pl.kernel(
        body,
        out_type=jax.ShapeDtypeStruct((NC, n_nodes, d), jnp.float32),
        mesh=mesh,
        scratch_types=[
            pltpu.VMEM((nchunk, CHUNK), jnp.int32),
            pltpu.VMEM((nchunk, CHUNK), jnp.int32),
            pltpu.VMEM((nchunk, CHUNK), jnp.float32),
            pltpu.VMEM((nbuf, CHUNK, d), jnp.float32),
            pltpu.VMEM_SHARED((n_nodes, d), jnp.float32),
        ] + [pltpu.SemaphoreType.DMA] * nbuf,
        compiler_params=pltpu.CompilerParams(use_tc_tiling_on_sc=False),
    )
    return f(support, esrc, etgt, ef)


def _matmul(x, w, bm=1000):
    n, kdim = x.shape
    m = w.shape[1]

    def mk(x_ref, w_ref, o_ref):
        o_ref[...] = jnp.dot(x_ref[...], w_ref[...],
                             preferred_element_type=jnp.float32)

    return pl.pallas_call(
        mk,
        grid=(n // bm,),
        in_specs=[pl.BlockSpec((bm, kdim), lambda i: (i, 0)),
                  pl.BlockSpec((kdim, m), lambda i: (0, 0))],
        out_specs=pl.BlockSpec((bm, m), lambda i: (i, 0)),
        out_shape=jax.ShapeDtypeStruct((n, m), jnp.float32),
    )(x, w)


def _norm_mm(p0, p1, b0, gamma, beta, x, w2, gm, gmt, bm=1000):
    n, c = x.shape
    m = w2.shape[1]
    inv_gs = float(GROUPS) / float(c)

    def fk(p0_ref, p1_ref, b0_ref, g_ref, be_ref, x_ref, w2_ref, gm_ref,
           gmt_ref, o_ref):
        t = jnp.maximum(p0_ref[...] + p1_ref[...] + b0_ref[...], 0.0)
        gmat = gm_ref[...]
        gmatt = gmt_ref[...]
        m32 = jnp.dot(t, gmat, preferred_element_type=jnp.float32) * inv_gs
        mf = jnp.dot(m32, gmatt, preferred_element_type=jnp.float32)
        dlt = t - mf
        v32 = jnp.dot(dlt * dlt, gmat,
                      preferred_element_type=jnp.float32) * inv_gs
        invf = jnp.dot(lax.rsqrt(v32 + EPS), gmatt,
                       preferred_element_type=jnp.float32)
        h = dlt * invf * g_ref[...] + be_ref[...] + x_ref[...]
        o_ref[...] = jnp.dot(h, w2_ref[...], preferred_element_type=jnp.float32)

    return pl.pallas_call(
        fk,
        grid=(n // bm,),
        in_specs=[pl.BlockSpec((bm, c), lambda i: (i, 0)),
                  pl.BlockSpec((bm, c), lambda i: (i, 0)),
                  pl.BlockSpec((1, c), lambda i: (0, 0)),
                  pl.BlockSpec((1, c), lambda i: (0, 0)),
                  pl.BlockSpec((1, c), lambda i: (0, 0)),
                  pl.BlockSpec((bm, c), lambda i: (i, 0)),
                  pl.BlockSpec((c, m), lambda i: (0, 0)),
                  pl.BlockSpec((c, GROUPS), lambda i: (0, 0)),
                  pl.BlockSpec((GROUPS, c), lambda i: (0, 0))],
        out_specs=pl.BlockSpec((bm, m), lambda i: (i, 0)),
        out_shape=jax.ShapeDtypeStruct((n, m), jnp.float32),
    )(p0, p1, b0, gamma, beta, x, w2, gm, gmt)


def _combine(p0, p1, b2, ncls, bm=1000):
    n, dpad = p0.shape

    def ck(p0_ref, p1_ref, b2_ref, o_ref):
        s = p0_ref[...] + p1_ref[...]
        o_ref[...] = s[:, :ncls] + b2_ref[...]

    return pl.pallas_call(
        ck,
        grid=(n // bm,),
        in_specs=[pl.BlockSpec((bm, dpad), lambda i: (i, 0)),
                  pl.BlockSpec((bm, dpad), lambda i: (i, 0)),
                  pl.BlockSpec((1, ncls), lambda i: (0, 0))],
        out_specs=pl.BlockSpec((bm, ncls), lambda i: (i, 0)),
        out_shape=jax.ShapeDtypeStruct((n, ncls), jnp.float32),
    )(p0, p1, b2)


def kernel(x, Esrc, Etgt, ef, W0, b0, gamma0, beta0, W2, b2):
    n, c = x.shape
    e = Esrc.shape[0]
    ncls = W2.shape[1]
    d2 = 48

    ch1, blk1 = 96, 15
    nchunk1 = -(-e // (ch1 * NW * blk1)) * blk1
    nblk1 = nchunk1 // blk1
    pad1 = nchunk1 * ch1 * NW - e
    esrc1 = jnp.pad(Esrc, (0, pad1)).reshape(NW, nblk1, blk1, ch1)
    etgt1 = jnp.pad(Etgt, (0, pad1)).reshape(NW, nblk1, blk1, ch1)
    ef1 = jnp.pad(ef[:, 0], (0, pad1)).reshape(NW, nblk1, blk1, ch1)

    nchunk2 = -(-e // (CHUNK * NW * 4)) * 4
    pad2 = nchunk2 * CHUNK * NW - e
    esrc2 = jnp.pad(Esrc, (0, pad2)).reshape(NW, nchunk2, CHUNK)
    etgt2 = jnp.pad(Etgt, (0, pad2)).reshape(NW, nchunk2, CHUNK)
    ef2 = jnp.pad(ef[:, 0], (0, pad2)).reshape(NW, nchunk2, CHUNK)

    w2p = jnp.pad(W2, ((0, 0), (0, d2 - ncls)))
    gm = jnp.repeat(jnp.eye(GROUPS, dtype=jnp.float32), c // GROUPS, axis=0)

    support = _matmul(x, W0)
    parts = _propagate(support, esrc1, etgt1, ef1, n, c, nblk1, blk1, ch1)
    support2 = _norm_mm(parts[0], parts[1], b0.reshape(1, c),
                        gamma0.reshape(1, c), beta0.reshape(1, c),
                        x, w2p, gm, gm.T)
    parts2 = _propagate_flat(support2, esrc2, etgt2, ef2, n, d2, nchunk2)
    return _combine(parts2[0], parts2[1], b2.reshape(1, ncls), ncls)
